# baseline (speedup 1.0000x reference)
import functools

import jax
import jax.numpy as jnp
from jax import lax
from jax.experimental import pallas as pl
from jax.experimental.pallas import tpu as pltpu
from jax.experimental.pallas import tpu_sc as plsc

B, M, D, HOPS = 32, 4096, 128, 3
BSZ = 4
K1 = 12
K2 = B - K1
NW = 32
RPW = M // NW
DG = D // 16
QUAD = 4


_GDN = lax.GatherDimensionNumbers(offset_dims=(), collapsed_slice_dims=(0,),
                                  start_index_map=(0,))


def _shuffle(x, idx):
    return lax.gather(x, idx[:, None], _GDN, slice_sizes=(1,),
                      mode=lax.GatherScatterMode.PROMISE_IN_BOUNDS)


def _dot_rows(aref, u, out_ref, n_rows):
    lane = lax.iota(jnp.int32, 16)

    def grp(gg, _):
        base = gg * 16
        vecs = []
        for k in range(16):
            rr = base + k
            prods = [aref[rr, pl.ds(16 * j, 16)] * u[j] for j in range(DG)]
            while len(prods) > 1:
                prods = [prods[i] + prods[i + 1]
                         for i in range(0, len(prods), 2)]
            vecs.append(prods[0])
        for k in (1, 2, 4, 8):
            m = (lane & k) != 0
            nxt = []
            for i in range(0, len(vecs), 2):
                x, y = vecs[i], vecs[i + 1]
                xp = x + _shuffle(x, lane ^ k)
                yp = y + _shuffle(y, lane ^ k)
                nxt.append(jnp.where(m, yp, xp))
            vecs = nxt
        out_ref[pl.ds(base, 16)] = vecs[0]
        return 0
    lax.fori_loop(0, n_rows // 16, grp, 0)


def _sc_body(q_hbm, m0_hbm, t0_hbm, qv, b0, b1, b2, b3, lsl,
             s0, s1, s2, s3):
    c = lax.axis_index("c")
    t = lax.axis_index("s")
    wid = t * 2 + c
    r0 = wid * RPW
    bufs = (b0, b1, b2, b3)
    sems = (s0, s1, s2, s3)
    pltpu.sync_copy(q_hbm, qv)

    def quad_body(i, _):
        descs = []
        for j in range(QUAD):
            bi = i * QUAD + j
            descs.append(pltpu.async_copy(
                m0_hbm.at[K1 + bi, pl.ds(r0, RPW)], bufs[j], sems[j]))
        for j in range(QUAD):
            bi = i * QUAD + j
            descs[j].wait()
            u = tuple(qv[K1 + bi, pl.ds(16 * g, 16)] for g in range(DG))
            _dot_rows(bufs[j], u, lsl, RPW)
            pltpu.sync_copy(lsl, t0_hbm.at[bi, pl.ds(r0, RPW)])
        return 0

    lax.fori_loop(0, K2 // QUAD, quad_body, 0)


def _sc_call(query_vector, m0):
    mesh = plsc.VectorSubcoreMesh(core_axis_name="c", subcore_axis_name="s",
                                  num_cores=2, num_subcores=16)
    f32 = jnp.float32
    return pl.kernel(
        _sc_body,
        out_type=jax.ShapeDtypeStruct((K2, M), f32),
        mesh=mesh,
        scratch_types=[
            pltpu.VMEM((B, D), f32),
            pltpu.VMEM((RPW, D), f32),
            pltpu.VMEM((RPW, D), f32),
            pltpu.VMEM((RPW, D), f32),
            pltpu.VMEM((RPW, D), f32),
            pltpu.VMEM((RPW,), f32),
            pltpu.SemaphoreType.DMA,
            pltpu.SemaphoreType.DMA,
            pltpu.SemaphoreType.DMA,
            pltpu.SemaphoreType.DMA,
        ],
    )(query_vector, m0)


def _logits(a, u, g):
    t = jax.lax.dot_general(u, a, (((1,), (1,)), ((), ())),
                            preferred_element_type=jnp.float32)
    return t * g


def _softmax_tail(l, g, a_next, u):
    e = jnp.exp(l - jnp.max(l, axis=1, keepdims=True))
    eg = e * g
    o = jax.lax.dot_general(eg, a_next, (((1,), (0,)), ((), ())),
                            preferred_element_type=jnp.float32)
    return u + o / jnp.sum(e, axis=1, keepdims=True)


def _finish(l):
    e = jnp.exp(l - jnp.max(l, axis=1, keepdims=True))
    return e / jnp.sum(e, axis=1, keepdims=True)


def _tc_full_body(q_ref, g_ref, m0_ref, m1_ref, m2_ref, soft_ref, logits_ref):
    for b in range(BSZ):
        u = q_ref[0, b][None, :]
        g = g_ref[0, b][None, :]
        a0, a1, a2 = m0_ref[b], m1_ref[b], m2_ref[b]
        u = _softmax_tail(_logits(a0, u, g), g, a1, u)
        u = _softmax_tail(_logits(a1, u, g), g, a2, u)
        l = _logits(a2, u, g)
        soft_ref[0, b] = _finish(l)[0]
        logits_ref[0, b] = l[0]


def _tc_t0_body(q_ref, g_ref, t0_ref, m1_ref, m2_ref, soft_ref, logits_ref):
    for b in range(BSZ):
        u = q_ref[0, b][None, :]
        g = g_ref[0, b][None, :]
        a1, a2 = m1_ref[b], m2_ref[b]
        l0 = t0_ref[0, b][None, :] * g
        u = _softmax_tail(l0, g, a1, u)
        u = _softmax_tail(_logits(a1, u, g), g, a2, u)
        l = _logits(a2, u, g)
        soft_ref[0, b] = _finish(l)[0]
        logits_ref[0, b] = l[0]


def _tc_call_full(query_vector, global_pointer, m0, m1, m2):
    out = pl.pallas_call(
        _tc_full_body,
        grid=(K1 // BSZ,),
        in_specs=[
            pl.BlockSpec((1, BSZ, D), lambda i: (i, 0, 0)),
            pl.BlockSpec((1, BSZ, M), lambda i: (i, 0, 0)),
            pl.BlockSpec((BSZ, M, D), lambda i: (i, 0, 0)),
            pl.BlockSpec((BSZ, M, D), lambda i: (i, 0, 0)),
            pl.BlockSpec((BSZ, M, D), lambda i: (i, 0, 0)),
        ],
        out_specs=[
            pl.BlockSpec((1, BSZ, M), lambda i: (i, 0, 0)),
            pl.BlockSpec((1, BSZ, M), lambda i: (i, 0, 0)),
        ],
        out_shape=[
            jax.ShapeDtypeStruct((K1 // BSZ, BSZ, M), jnp.float32),
            jax.ShapeDtypeStruct((K1 // BSZ, BSZ, M), jnp.float32),
        ],
    )(query_vector[:K1].reshape(K1 // BSZ, BSZ, D),
      global_pointer[:K1].reshape(K1 // BSZ, BSZ, M), m0, m1, m2)
    return out[0].reshape(K1, M), out[1].reshape(K1, M)


def _tc_call_t0(query_vector, global_pointer, t0, m1, m2):
    off = K1 // BSZ
    out = pl.pallas_call(
        _tc_t0_body,
        grid=(K2 // BSZ,),
        in_specs=[
            pl.BlockSpec((1, BSZ, D), lambda i: (i, 0, 0)),
            pl.BlockSpec((1, BSZ, M), lambda i: (i, 0, 0)),
            pl.BlockSpec((1, BSZ, M), lambda i: (i, 0, 0)),
            pl.BlockSpec((BSZ, M, D), lambda i: (i + off, 0, 0)),
            pl.BlockSpec((BSZ, M, D), lambda i: (i + off, 0, 0)),
        ],
        out_specs=[
            pl.BlockSpec((1, BSZ, M), lambda i: (i, 0, 0)),
            pl.BlockSpec((1, BSZ, M), lambda i: (i, 0, 0)),
        ],
        out_shape=[
            jax.ShapeDtypeStruct((K2 // BSZ, BSZ, M), jnp.float32),
            jax.ShapeDtypeStruct((K2 // BSZ, BSZ, M), jnp.float32),
        ],
    )(query_vector[K1:].reshape(K2 // BSZ, BSZ, D),
      global_pointer[K1:].reshape(K2 // BSZ, BSZ, M),
      t0.reshape(K2 // BSZ, BSZ, M), m1, m2)
    return out[0].reshape(K2, M), out[1].reshape(K2, M)


@jax.jit
def kernel(query_vector, global_pointer, m0, m1, m2, m3):
    del m3
    t0 = _sc_call(query_vector, m0)
    tc1_soft, tc1_logits = _tc_call_full(query_vector, global_pointer,
                                         m0, m1, m2)
    tc2_soft, tc2_logits = _tc_call_t0(query_vector, global_pointer,
                                       t0, m1, m2)
    return (jnp.concatenate([tc1_soft, tc2_soft], axis=0),
            jnp.concatenate([tc1_logits, tc2_logits], axis=0))


# speedup vs baseline: 1.3671x; 1.3671x over previous
import jax
import jax.numpy as jnp
from jax.experimental import pallas as pl

B, M, D, HOPS = 32, 4096, 128, 3
BSZ = 4


def _logits(a, u, g):
    t = jax.lax.dot_general(u, a, (((1,), (1,)), ((), ())),
                            preferred_element_type=jnp.float32)
    return t * g


def _body(q_ref, g_ref, m0_ref, m1_ref, m2_ref, soft_ref, logits_ref):
    for b in range(BSZ):
        u = q_ref[0, b][None, :]
        g = g_ref[0, b][None, :]
        a0 = m0_ref[b]
        a1 = m1_ref[b]
        a2 = m2_ref[b]

        def hop(a_logits, a_next, u, g):
            l = _logits(a_logits, u, g)
            e = jnp.exp(l - jnp.max(l, axis=1, keepdims=True))
            eg = e * g
            o = jax.lax.dot_general(eg, a_next, (((1,), (0,)), ((), ())),
                                    preferred_element_type=jnp.float32)
            return u + o / jnp.sum(e, axis=1, keepdims=True)

        u = hop(a0, a1, u, g)
        u = hop(a1, a2, u, g)
        l = _logits(a2, u, g)
        e = jnp.exp(l - jnp.max(l, axis=1, keepdims=True))
        p = e / jnp.sum(e, axis=1, keepdims=True)
        soft_ref[0, b] = p[0]
        logits_ref[0, b] = l[0]


@jax.jit
def kernel(query_vector, global_pointer, m0, m1, m2, m3):
    del m3
    out = pl.pallas_call(
        _body,
        grid=(B // BSZ,),
        in_specs=[
            pl.BlockSpec((1, BSZ, D), lambda i: (i, 0, 0)),
            pl.BlockSpec((1, BSZ, M), lambda i: (i, 0, 0)),
            pl.BlockSpec((BSZ, M, D), lambda i: (i, 0, 0)),
            pl.BlockSpec((BSZ, M, D), lambda i: (i, 0, 0)),
            pl.BlockSpec((BSZ, M, D), lambda i: (i, 0, 0)),
        ],
        out_specs=[
            pl.BlockSpec((1, BSZ, M), lambda i: (i, 0, 0)),
            pl.BlockSpec((1, BSZ, M), lambda i: (i, 0, 0)),
        ],
        out_shape=[
            jax.ShapeDtypeStruct((B // BSZ, BSZ, M), jnp.float32),
            jax.ShapeDtypeStruct((B // BSZ, BSZ, M), jnp.float32),
        ],
    )(query_vector.reshape(B // BSZ, BSZ, D),
      global_pointer.reshape(B // BSZ, BSZ, M), m0, m1, m2)
    return (out[0].reshape(B, M), out[1].reshape(B, M))
